# Initial kernel scaffold; baseline (speedup 1.0000x reference)
import jax
import jax.numpy as jnp
from jax.experimental import pallas as pl


def kernel(x, edge_index, edge_attr, W10, W11, b1, W20, W21, b2, L1w, L1b, L2w, L2b):
    raise NotImplementedError("write your pallas kernel here")


# speedup vs baseline: 13.2331x; 13.2331x over previous
import jax
import jax.numpy as jnp
from jax import lax
from jax.experimental import pallas as pl
from jax.experimental.pallas import tpu as pltpu
from jax.experimental.pallas import tpu_sc as plsc

N = 10000
E = 160000
D = 256
P = 12
F = 64
NPAD = 10240
EP = 163840
K = 128
ECH = EP // K
NC = 2
NS = 16
RT = NPAD // NS
ER4 = ECH // NS
ER1 = ECH // (NC * NS)

_MESH = dict(core_axis_name="c", subcore_axis_name="s")
_SC_PARAMS = pltpu.CompilerParams(use_tc_tiling_on_sc=False,
                                  needs_layout_passes=False)


def _zero_2d(buf, rows, vecs):
    def body(r, _):
        for k in range(vecs):
            buf[r, pl.ds(k * 16, 16)] = jnp.zeros((16,), jnp.float32)
        return 0
    lax.fori_loop(0, rows, body, 0)


def _norm_inplace(dis_t, src_t, dst_t, norm_t, nrows):
    g = K // 16

    def body(t, _):
        j = t // g
        o = (t % g) * 16
        s16 = plsc.load_gather(dis_t, [src_t[j, pl.ds(o, 16)]])
        d16 = plsc.load_gather(dis_t, [dst_t[j, pl.ds(o, 16)]])
        ew16 = norm_t[j, pl.ds(o, 16)]
        norm_t[j, pl.ds(o, 16)] = -(s16 * d16 * ew16)
        return 0
    lax.fori_loop(0, nrows * g, body, 0)


def _scale_rows(gbuf, norm_t, ch, vecs):
    def body(g, _):
        nvec = norm_t[ch, pl.ds(g * 16, 16)]
        for l in range(16):
            nv = jnp.broadcast_to(nvec[l], (16,))
            e = g * 16 + l
            for k in range(vecs):
                gbuf[e, pl.ds(k * 16, 16)] = gbuf[e, pl.ds(k * 16, 16)] * nv
        return 0
    lax.fori_loop(0, K // 16, body, 0)


def _deg_body(src_h, ew_h, out_h, src_t, ew_t, zb, deg_sh):
    c = lax.axis_index("c")
    w = lax.axis_index("s")
    row0 = (c * NS + w) * ER1
    pltpu.sync_copy(src_h.at[pl.ds(row0, ER1)], src_t)
    pltpu.sync_copy(ew_h.at[pl.ds(row0, ER1)], ew_t)

    def zbody(r, _):
        zb[pl.ds(r * 16, 16)] = jnp.zeros((16,), jnp.float32)
        return 0
    lax.fori_loop(0, RT // 16, zbody, 0)
    pltpu.sync_copy(zb, deg_sh.at[pl.ds(w * RT, RT)])
    plsc.subcore_barrier()

    def chunk_body(j, _):
        pltpu.sync_copy(ew_t.at[j], deg_sh.at[src_t.at[j]], add=True)
        return 0
    lax.fori_loop(0, ER1, chunk_body, 0)
    plsc.subcore_barrier()
    pltpu.sync_copy(deg_sh.at[pl.ds(w * RT, RT)], out_h.at[c, pl.ds(w * RT, RT)])


def _sc_deg(src2, ew2):
    kfn = pl.kernel(
        _deg_body,
        out_type=jax.ShapeDtypeStruct((NC, NPAD), jnp.float32),
        mesh=plsc.VectorSubcoreMesh(**_MESH),
        compiler_params=_SC_PARAMS,
        scratch_types=[
            pltpu.VMEM((ER1, K), jnp.int32),
            pltpu.VMEM((ER1, K), jnp.float32),
            pltpu.VMEM((RT,), jnp.float32),
            pltpu.VMEM_SHARED((NPAD,), jnp.float32),
        ],
    )
    return kfn(src2, ew2)


def _spmm_body(u12_h, src_h, dst_h, ew_h, dis_h, out_h,
               dis_t, idx_t, dst_t, norm_t, gbuf, z_sh):
    c = lax.axis_index("c")
    w = lax.axis_index("s")
    row0 = w * ER4
    pltpu.sync_copy(dis_h, dis_t)
    pltpu.sync_copy(src_h.at[pl.ds(row0, ER4)], idx_t)
    pltpu.sync_copy(dst_h.at[pl.ds(row0, ER4)], dst_t)
    pltpu.sync_copy(ew_h.at[pl.ds(row0, ER4)], norm_t)
    _norm_inplace(dis_t, idx_t, dst_t, norm_t, ER4)

    def to_idx(t, _):
        j = t // (K // 16)
        o = (t % (K // 16)) * 16
        idx_t[j, pl.ds(o, 16)] = idx_t[j, pl.ds(o, 16)] * P + c * (P // NC)
        return 0
    lax.fori_loop(0, ER4 * (K // 16), to_idx, 0)

    _zero_2d(gbuf, K, F // 16)
    for i in range(RT // K):
        pltpu.sync_copy(gbuf, z_sh.at[pl.ds(w * RT + i * K, K)])
    plsc.subcore_barrier()

    for jp in range(P // NC):
        p = c * (P // NC) + jp

        def chunk_body(ch, _):
            pltpu.sync_copy(u12_h.at[idx_t.at[ch]], gbuf)
            _scale_rows(gbuf, norm_t, ch, F // 16)
            pltpu.sync_copy(gbuf, z_sh.at[dst_t.at[ch]], add=True)
            return 0

        lax.fori_loop(0, ER4, chunk_body, 0)
        plsc.subcore_barrier()
        pltpu.sync_copy(z_sh.at[pl.ds(w * RT, RT)],
                        out_h.at[p, pl.ds(w * RT, RT)])
        if jp < P // NC - 1:
            def advance(t, _):
                j = t // (K // 16)
                o = (t % (K // 16)) * 16
                idx_t[j, pl.ds(o, 16)] = idx_t[j, pl.ds(o, 16)] + 1
                return 0
            lax.fori_loop(0, ER4 * (K // 16), advance, 0)
            _zero_2d(gbuf, K, F // 16)
            for i in range(RT // K):
                pltpu.sync_copy(gbuf, z_sh.at[pl.ds(w * RT + i * K, K)])
            plsc.subcore_barrier()


def _sc_spmm_panels(u12, src2, dst2, ew2, dis):
    kfn = pl.kernel(
        _spmm_body,
        out_type=jax.ShapeDtypeStruct((P, NPAD, F), jnp.float32),
        mesh=plsc.VectorSubcoreMesh(**_MESH),
        compiler_params=_SC_PARAMS,
        scratch_types=[
            pltpu.VMEM((NPAD,), jnp.float32),
            pltpu.VMEM((ER4, K), jnp.int32),
            pltpu.VMEM((ER4, K), jnp.int32),
            pltpu.VMEM((ER4, K), jnp.float32),
            pltpu.VMEM((K, F), jnp.float32),
            pltpu.VMEM_SHARED((NPAD, F), jnp.float32),
        ],
    )
    return kfn(u12, src2, dst2, ew2, dis)


def _spmm1_body(s_h, src_h, dst_h, ew_h, dis_h, out_h,
                dis_t, src_t, dst_t, norm_t, gbuf, t_sh):
    c = lax.axis_index("c")
    w = lax.axis_index("s")
    row0 = (c * NS + w) * ER1
    pltpu.sync_copy(dis_h, dis_t)
    pltpu.sync_copy(src_h.at[pl.ds(row0, ER1)], src_t)
    pltpu.sync_copy(dst_h.at[pl.ds(row0, ER1)], dst_t)
    pltpu.sync_copy(ew_h.at[pl.ds(row0, ER1)], norm_t)
    _norm_inplace(dis_t, src_t, dst_t, norm_t, ER1)

    _zero_2d(gbuf, K, F // 16)
    for i in range(RT // K):
        pltpu.sync_copy(gbuf, t_sh.at[pl.ds(w * RT + i * K, K)])
    plsc.subcore_barrier()

    def chunk_body(ch, _):
        pltpu.sync_copy(s_h.at[src_t.at[ch]], gbuf)
        _scale_rows(gbuf, norm_t, ch, F // 16)
        pltpu.sync_copy(gbuf, t_sh.at[dst_t.at[ch]], add=True)
        return 0

    lax.fori_loop(0, ER1, chunk_body, 0)
    plsc.subcore_barrier()
    pltpu.sync_copy(t_sh.at[pl.ds(w * RT, RT)],
                    out_h.at[c, pl.ds(w * RT, RT)])


def _sc_spmm_single(s, src2, dst2, ew2, dis):
    kfn = pl.kernel(
        _spmm1_body,
        out_type=jax.ShapeDtypeStruct((NC, NPAD, F), jnp.float32),
        mesh=plsc.VectorSubcoreMesh(**_MESH),
        compiler_params=_SC_PARAMS,
        scratch_types=[
            pltpu.VMEM((NPAD,), jnp.float32),
            pltpu.VMEM((ER1, K), jnp.int32),
            pltpu.VMEM((ER1, K), jnp.int32),
            pltpu.VMEM((ER1, K), jnp.float32),
            pltpu.VMEM((K, F), jnp.float32),
            pltpu.VMEM_SHARED((NPAD, F), jnp.float32),
        ],
    )
    return kfn(s, src2, dst2, ew2, dis)


NB = 1000


def _k1_body(xt_ref, wc_ref, v_ref, u_ref):
    wc = wc_ref[...]
    for p in range(P):
        xp = xt_ref[p]
        y = lax.dot_general(xp, wc, (((1,), (1,)), ((), ())),
                            preferred_element_type=jnp.float32)
        v_ref[:, p, :] = y[:, :F]
        u_ref[:, p, :] = y[:, F:]


def _tc_proj(xt, wc):
    return pl.pallas_call(
        _k1_body,
        grid=(N // NB,),
        in_specs=[
            pl.BlockSpec((P, NB, D), lambda i: (0, i, 0)),
            pl.BlockSpec((2 * F, D), lambda i: (0, 0)),
        ],
        out_specs=[
            pl.BlockSpec((NB, P, F), lambda i: (i, 0, 0)),
            pl.BlockSpec((NB, P, F), lambda i: (i, 0, 0)),
        ],
        out_shape=[
            jax.ShapeDtypeStruct((N, P, F), jnp.float32),
            jax.ShapeDtypeStruct((N, P, F), jnp.float32),
        ],
    )(xt, wc)


def _k3_body(deg_ref, dis_ref):
    d = deg_ref[0:1, :] + deg_ref[1:2, :]
    pos = d > 0
    dis_ref[...] = jnp.where(pos, lax.rsqrt(jnp.where(pos, d, 1.0)), 0.0)


def _tc_dis(deg2):
    return pl.pallas_call(
        _k3_body,
        out_shape=jax.ShapeDtypeStruct((1, NPAD), jnp.float32),
    )(deg2)


def _k5_body(v_ref, z_ref, b1_ref, s_ref):
    b1 = b1_ref[...]
    acc = jnp.zeros((NB, F), jnp.float32)
    for p in range(P):
        acc = acc + jax.nn.relu(v_ref[:, p, :] + z_ref[p] + b1)
    s_ref[...] = acc


def _tc_sum(v3, z12, b1):
    return pl.pallas_call(
        _k5_body,
        grid=(N // NB,),
        in_specs=[
            pl.BlockSpec((NB, P, F), lambda i: (i, 0, 0)),
            pl.BlockSpec((P, NB, F), lambda i: (0, i, 0)),
            pl.BlockSpec((1, F), lambda i: (0, 0)),
        ],
        out_specs=pl.BlockSpec((NB, F), lambda i: (i, 0)),
        out_shape=jax.ShapeDtypeStruct((N, F), jnp.float32),
    )(v3, z12, b1)


def _k7_body(s_ref, t_ref, w20_ref, w21_ref, b2_ref, l1w_ref, l1b_ref,
             l2w_ref, l2b_ref, h_ref, y_ref):
    s = s_ref[...]
    t = t_ref[0] + t_ref[1]
    hh = (lax.dot_general(s, w20_ref[...], (((1,), (1,)), ((), ())),
                          preferred_element_type=jnp.float32)
          + lax.dot_general(t, w21_ref[...], (((1,), (1,)), ((), ())),
                            preferred_element_type=jnp.float32)
          + P * b2_ref[...])
    a1 = jax.nn.relu(
        lax.dot_general(hh, l1w_ref[...], (((1,), (1,)), ((), ())),
                        preferred_element_type=jnp.float32) + l1b_ref[...])
    y = lax.dot_general(a1, l2w_ref[...], (((1,), (1,)), ((), ())),
                        preferred_element_type=jnp.float32) + l2b_ref[...]
    h_ref[...] = hh
    y_ref[...] = y


def _tc_head(s, t2, w20, w21, b2, l1w, l1b, l2w, l2b):
    def full(shape):
        return pl.BlockSpec(shape, lambda i, _s=shape: tuple(0 for _ in _s))
    return pl.pallas_call(
        _k7_body,
        grid=(N // NB,),
        in_specs=[
            pl.BlockSpec((NB, F), lambda i: (i, 0)),
            pl.BlockSpec((NC, NB, F), lambda i: (0, i, 0)),
            full((D, F)), full((D, F)), full((1, D)),
            full((128, D)), full((1, 128)),
            full((P, 128)), full((1, P)),
        ],
        out_specs=[
            pl.BlockSpec((NB, D), lambda i: (i, 0)),
            pl.BlockSpec((NB, P), lambda i: (i, 0)),
        ],
        out_shape=[
            jax.ShapeDtypeStruct((N, D), jnp.float32),
            jax.ShapeDtypeStruct((N, P), jnp.float32),
        ],
    )(s, t2, w20, w21, b2, l1w, l1b, l2w, l2b)


@jax.jit
def kernel(x, edge_index, edge_attr, W10, W11, b1, W20, W21, b2, L1w, L1b, L2w, L2b):
    src = edge_index[0].astype(jnp.int32)
    dst = edge_index[1].astype(jnp.int32)
    pad = EP - E
    pidx = (jnp.arange(pad, dtype=jnp.int32) % N)
    src2 = jnp.concatenate([src, pidx]).reshape(ECH, K)
    dst2 = jnp.concatenate([dst, pidx]).reshape(ECH, K)
    ew2 = jnp.concatenate([edge_attr, jnp.zeros((pad,), jnp.float32)]).reshape(ECH, K)

    xt = jnp.transpose(x, (2, 0, 1))
    wc = jnp.concatenate([W10, W11], axis=0)
    v3, u3 = _tc_proj(xt, wc)
    u12 = u3.reshape(P * N, F)

    deg2 = _sc_deg(src2, ew2)
    dis = _tc_dis(deg2).reshape(NPAD)

    z12 = _sc_spmm_panels(u12, src2, dst2, ew2, dis)
    s = _tc_sum(v3, z12, b1.reshape(1, F))
    t2 = _sc_spmm_single(s, src2, dst2, ew2, dis)
    h, y = _tc_head(s, t2, W20, W21, b2.reshape(1, D),
                    L1w, L1b.reshape(1, 128), L2w, L2b.reshape(1, P))
    return (y, h)


# speedup vs baseline: 19.4730x; 1.4715x over previous
import jax
import jax.numpy as jnp
from jax import lax
from jax.experimental import pallas as pl
from jax.experimental.pallas import tpu as pltpu
from jax.experimental.pallas import tpu_sc as plsc

N = 10000
E = 160000
D = 256
P = 12
F = 64
NPAD = 10240
EP = 163840
K = 256
ECH = EP // K
NC = 2
NS = 16
RT = NPAD // NS
ER4 = ECH // NS
ER1 = ECH // (NC * NS)

_MESH = dict(core_axis_name="c", subcore_axis_name="s")
_SC_PARAMS = pltpu.CompilerParams(use_tc_tiling_on_sc=False,
                                  needs_layout_passes=False)


def _zero_2d(buf, rows, vecs):
    def body(r, _):
        for k in range(vecs):
            buf[r, pl.ds(k * 16, 16)] = jnp.zeros((16,), jnp.float32)
        return 0
    lax.fori_loop(0, rows, body, 0)


def _norm_inplace(dis_t, src_t, dst_t, norm_t, nrows):
    g = K // 16

    def body(t, _):
        j = t // g
        o = (t % g) * 16
        s16 = plsc.load_gather(dis_t, [src_t[j, pl.ds(o, 16)]])
        d16 = plsc.load_gather(dis_t, [dst_t[j, pl.ds(o, 16)]])
        ew16 = norm_t[j, pl.ds(o, 16)]
        norm_t[j, pl.ds(o, 16)] = -(s16 * d16 * ew16)
        return 0
    lax.fori_loop(0, nrows * g, body, 0)


def _scale_rows(gbuf, norm_t, ch, vecs):
    def body(g, _):
        nvec = norm_t[ch, pl.ds(g * 16, 16)]
        for l in range(16):
            nv = jnp.broadcast_to(nvec[l], (16,))
            e = g * 16 + l
            for k in range(vecs):
                gbuf[e, pl.ds(k * 16, 16)] = gbuf[e, pl.ds(k * 16, 16)] * nv
        return 0
    lax.fori_loop(0, K // 16, body, 0)


def _deg_body(src_h, ew_h, out_h, src_t, ew_t, zb, deg_sh):
    c = lax.axis_index("c")
    w = lax.axis_index("s")
    row0 = (c * NS + w) * ER1
    pltpu.sync_copy(src_h.at[pl.ds(row0, ER1)], src_t)
    pltpu.sync_copy(ew_h.at[pl.ds(row0, ER1)], ew_t)

    def zbody(r, _):
        zb[pl.ds(r * 16, 16)] = jnp.zeros((16,), jnp.float32)
        return 0
    lax.fori_loop(0, RT // 16, zbody, 0)
    pltpu.sync_copy(zb, deg_sh.at[pl.ds(w * RT, RT)])
    plsc.subcore_barrier()

    def chunk_body(j, _):
        pltpu.sync_copy(ew_t.at[j], deg_sh.at[src_t.at[j]], add=True)
        return 0
    lax.fori_loop(0, ER1, chunk_body, 0)
    plsc.subcore_barrier()
    pltpu.sync_copy(deg_sh.at[pl.ds(w * RT, RT)], out_h.at[c, pl.ds(w * RT, RT)])


def _sc_deg(src2, ew2):
    kfn = pl.kernel(
        _deg_body,
        out_type=jax.ShapeDtypeStruct((NC, NPAD), jnp.float32),
        mesh=plsc.VectorSubcoreMesh(**_MESH),
        compiler_params=_SC_PARAMS,
        scratch_types=[
            pltpu.VMEM((ER1, K), jnp.int32),
            pltpu.VMEM((ER1, K), jnp.float32),
            pltpu.VMEM((RT,), jnp.float32),
            pltpu.VMEM_SHARED((NPAD,), jnp.float32),
        ],
    )
    return kfn(src2, ew2)


def _spmm_body(u12_h, src_h, dst_h, ew_h, dis_h, out_h,
               dis_t, idx_t, dst_t, norm_t, gba, gbb, sema, semb, z_sh):
    c = lax.axis_index("c")
    w = lax.axis_index("s")
    row0 = w * ER4
    pltpu.sync_copy(dis_h, dis_t)
    pltpu.sync_copy(src_h.at[pl.ds(row0, ER4)], idx_t)
    pltpu.sync_copy(dst_h.at[pl.ds(row0, ER4)], dst_t)
    pltpu.sync_copy(ew_h.at[pl.ds(row0, ER4)], norm_t)
    _norm_inplace(dis_t, idx_t, dst_t, norm_t, ER4)

    def to_idx(t, _):
        j = t // (K // 16)
        o = (t % (K // 16)) * 16
        idx_t[j, pl.ds(o, 16)] = idx_t[j, pl.ds(o, 16)] * P + c * (P // NC)
        return 0
    lax.fori_loop(0, ER4 * (K // 16), to_idx, 0)

    _zero_2d(gba, K, F // 16)
    for i in range(RT // K):
        pltpu.sync_copy(gba, z_sh.at[pl.ds(w * RT + i * K, K)])
    plsc.subcore_barrier()

    for jp in range(P // NC):
        p = c * (P // NC) + jp
        pltpu.async_copy(u12_h.at[idx_t.at[0]], gba, sema)

        def pair_body(i, _):
            c0 = 2 * i
            c1 = 2 * i + 1
            pltpu.async_copy(u12_h.at[idx_t.at[c1]], gbb, semb)
            pltpu.make_async_copy(u12_h.at[idx_t.at[c0]], gba, sema).wait()
            _scale_rows(gba, norm_t, c0, F // 16)
            pltpu.sync_copy(gba, z_sh.at[dst_t.at[c0]], add=True)

            @pl.when(i < ER4 // 2 - 1)
            def _():
                pltpu.async_copy(u12_h.at[idx_t.at[c0 + 2]], gba, sema)
            pltpu.make_async_copy(u12_h.at[idx_t.at[c1]], gbb, semb).wait()
            _scale_rows(gbb, norm_t, c1, F // 16)
            pltpu.sync_copy(gbb, z_sh.at[dst_t.at[c1]], add=True)
            return 0

        lax.fori_loop(0, ER4 // 2, pair_body, 0)
        plsc.subcore_barrier()
        pltpu.sync_copy(z_sh.at[pl.ds(w * RT, RT)],
                        out_h.at[p, pl.ds(w * RT, RT)])
        if jp < P // NC - 1:
            def advance(t, _):
                j = t // (K // 16)
                o = (t % (K // 16)) * 16
                idx_t[j, pl.ds(o, 16)] = idx_t[j, pl.ds(o, 16)] + 1
                return 0
            lax.fori_loop(0, ER4 * (K // 16), advance, 0)
            _zero_2d(gba, K, F // 16)
            for i in range(RT // K):
                pltpu.sync_copy(gba, z_sh.at[pl.ds(w * RT + i * K, K)])
            plsc.subcore_barrier()


def _sc_spmm_panels(u12, src2, dst2, ew2, dis):
    kfn = pl.kernel(
        _spmm_body,
        out_type=jax.ShapeDtypeStruct((P, NPAD, F), jnp.float32),
        mesh=plsc.VectorSubcoreMesh(**_MESH),
        compiler_params=_SC_PARAMS,
        scratch_types=[
            pltpu.VMEM((NPAD,), jnp.float32),
            pltpu.VMEM((ER4, K), jnp.int32),
            pltpu.VMEM((ER4, K), jnp.int32),
            pltpu.VMEM((ER4, K), jnp.float32),
            pltpu.VMEM((K, F), jnp.float32),
            pltpu.VMEM((K, F), jnp.float32),
            pltpu.SemaphoreType.DMA,
            pltpu.SemaphoreType.DMA,
            pltpu.VMEM_SHARED((NPAD, F), jnp.float32),
        ],
    )
    return kfn(u12, src2, dst2, ew2, dis)


def _spmm1_body(s_h, src_h, dst_h, ew_h, dis_h, out_h,
                dis_t, src_t, dst_t, norm_t, gba, gbb, sema, semb, t_sh):
    c = lax.axis_index("c")
    w = lax.axis_index("s")
    row0 = (c * NS + w) * ER1
    pltpu.sync_copy(dis_h, dis_t)
    pltpu.sync_copy(src_h.at[pl.ds(row0, ER1)], src_t)
    pltpu.sync_copy(dst_h.at[pl.ds(row0, ER1)], dst_t)
    pltpu.sync_copy(ew_h.at[pl.ds(row0, ER1)], norm_t)
    _norm_inplace(dis_t, src_t, dst_t, norm_t, ER1)

    _zero_2d(gba, K, F // 16)
    for i in range(RT // K):
        pltpu.sync_copy(gba, t_sh.at[pl.ds(w * RT + i * K, K)])
    plsc.subcore_barrier()
    pltpu.async_copy(s_h.at[src_t.at[0]], gba, sema)

    def pair_body(i, _):
        c0 = 2 * i
        c1 = 2 * i + 1
        pltpu.async_copy(s_h.at[src_t.at[c1]], gbb, semb)
        pltpu.make_async_copy(s_h.at[src_t.at[c0]], gba, sema).wait()
        _scale_rows(gba, norm_t, c0, F // 16)
        pltpu.sync_copy(gba, t_sh.at[dst_t.at[c0]], add=True)

        @pl.when(i < ER1 // 2 - 1)
        def _():
            pltpu.async_copy(s_h.at[src_t.at[c0 + 2]], gba, sema)
        pltpu.make_async_copy(s_h.at[src_t.at[c1]], gbb, semb).wait()
        _scale_rows(gbb, norm_t, c1, F // 16)
        pltpu.sync_copy(gbb, t_sh.at[dst_t.at[c1]], add=True)
        return 0

    lax.fori_loop(0, ER1 // 2, pair_body, 0)
    plsc.subcore_barrier()
    pltpu.sync_copy(t_sh.at[pl.ds(w * RT, RT)],
                    out_h.at[c, pl.ds(w * RT, RT)])


def _sc_spmm_single(s, src2, dst2, ew2, dis):
    kfn = pl.kernel(
        _spmm1_body,
        out_type=jax.ShapeDtypeStruct((NC, NPAD, F), jnp.float32),
        mesh=plsc.VectorSubcoreMesh(**_MESH),
        compiler_params=_SC_PARAMS,
        scratch_types=[
            pltpu.VMEM((NPAD,), jnp.float32),
            pltpu.VMEM((ER1, K), jnp.int32),
            pltpu.VMEM((ER1, K), jnp.int32),
            pltpu.VMEM((ER1, K), jnp.float32),
            pltpu.VMEM((K, F), jnp.float32),
            pltpu.VMEM((K, F), jnp.float32),
            pltpu.SemaphoreType.DMA,
            pltpu.SemaphoreType.DMA,
            pltpu.VMEM_SHARED((NPAD, F), jnp.float32),
        ],
    )
    return kfn(s, src2, dst2, ew2, dis)


NB = 1000


def _k1_body(xt_ref, wc_ref, v_ref, u_ref):
    wc = wc_ref[...]
    for p in range(P):
        xp = xt_ref[p]
        y = lax.dot_general(xp, wc, (((1,), (1,)), ((), ())),
                            preferred_element_type=jnp.float32)
        v_ref[:, p, :] = y[:, :F]
        u_ref[:, p, :] = y[:, F:]


def _tc_proj(xt, wc):
    return pl.pallas_call(
        _k1_body,
        grid=(N // NB,),
        in_specs=[
            pl.BlockSpec((P, NB, D), lambda i: (0, i, 0)),
            pl.BlockSpec((2 * F, D), lambda i: (0, 0)),
        ],
        out_specs=[
            pl.BlockSpec((NB, P, F), lambda i: (i, 0, 0)),
            pl.BlockSpec((NB, P, F), lambda i: (i, 0, 0)),
        ],
        out_shape=[
            jax.ShapeDtypeStruct((N, P, F), jnp.float32),
            jax.ShapeDtypeStruct((N, P, F), jnp.float32),
        ],
    )(xt, wc)


def _k3_body(deg_ref, dis_ref):
    d = deg_ref[0:1, :] + deg_ref[1:2, :]
    pos = d > 0
    dis_ref[...] = jnp.where(pos, lax.rsqrt(jnp.where(pos, d, 1.0)), 0.0)


def _tc_dis(deg2):
    return pl.pallas_call(
        _k3_body,
        out_shape=jax.ShapeDtypeStruct((1, NPAD), jnp.float32),
    )(deg2)


def _k5_body(v_ref, z_ref, b1_ref, s_ref):
    b1 = b1_ref[...]
    acc = jnp.zeros((NB, F), jnp.float32)
    for p in range(P):
        acc = acc + jax.nn.relu(v_ref[:, p, :] + z_ref[p] + b1)
    s_ref[...] = acc


def _tc_sum(v3, z12, b1):
    return pl.pallas_call(
        _k5_body,
        grid=(N // NB,),
        in_specs=[
            pl.BlockSpec((NB, P, F), lambda i: (i, 0, 0)),
            pl.BlockSpec((P, NB, F), lambda i: (0, i, 0)),
            pl.BlockSpec((1, F), lambda i: (0, 0)),
        ],
        out_specs=pl.BlockSpec((NB, F), lambda i: (i, 0)),
        out_shape=jax.ShapeDtypeStruct((N, F), jnp.float32),
    )(v3, z12, b1)


def _k7_body(s_ref, t_ref, w20_ref, w21_ref, b2_ref, l1w_ref, l1b_ref,
             l2w_ref, l2b_ref, h_ref, y_ref):
    s = s_ref[...]
    t = t_ref[0] + t_ref[1]
    hh = (lax.dot_general(s, w20_ref[...], (((1,), (1,)), ((), ())),
                          preferred_element_type=jnp.float32)
          + lax.dot_general(t, w21_ref[...], (((1,), (1,)), ((), ())),
                            preferred_element_type=jnp.float32)
          + P * b2_ref[...])
    a1 = jax.nn.relu(
        lax.dot_general(hh, l1w_ref[...], (((1,), (1,)), ((), ())),
                        preferred_element_type=jnp.float32) + l1b_ref[...])
    y = lax.dot_general(a1, l2w_ref[...], (((1,), (1,)), ((), ())),
                        preferred_element_type=jnp.float32) + l2b_ref[...]
    h_ref[...] = hh
    y_ref[...] = y


def _tc_head(s, t2, w20, w21, b2, l1w, l1b, l2w, l2b):
    def full(shape):
        return pl.BlockSpec(shape, lambda i, _s=shape: tuple(0 for _ in _s))
    return pl.pallas_call(
        _k7_body,
        grid=(N // NB,),
        in_specs=[
            pl.BlockSpec((NB, F), lambda i: (i, 0)),
            pl.BlockSpec((NC, NB, F), lambda i: (0, i, 0)),
            full((D, F)), full((D, F)), full((1, D)),
            full((128, D)), full((1, 128)),
            full((P, 128)), full((1, P)),
        ],
        out_specs=[
            pl.BlockSpec((NB, D), lambda i: (i, 0)),
            pl.BlockSpec((NB, P), lambda i: (i, 0)),
        ],
        out_shape=[
            jax.ShapeDtypeStruct((N, D), jnp.float32),
            jax.ShapeDtypeStruct((N, P), jnp.float32),
        ],
    )(s, t2, w20, w21, b2, l1w, l1b, l2w, l2b)


@jax.jit
def kernel(x, edge_index, edge_attr, W10, W11, b1, W20, W21, b2, L1w, L1b, L2w, L2b):
    src = edge_index[0].astype(jnp.int32)
    dst = edge_index[1].astype(jnp.int32)
    pad = EP - E
    pidx = (jnp.arange(pad, dtype=jnp.int32) % N)
    src2 = jnp.concatenate([src, pidx]).reshape(ECH, K)
    dst2 = jnp.concatenate([dst, pidx]).reshape(ECH, K)
    ew2 = jnp.concatenate([edge_attr, jnp.zeros((pad,), jnp.float32)]).reshape(ECH, K)

    xt = jnp.transpose(x, (2, 0, 1))
    wc = jnp.concatenate([W10, W11], axis=0)
    v3, u3 = _tc_proj(xt, wc)
    u12 = u3.reshape(P * N, F)

    deg2 = _sc_deg(src2, ew2)
    dis = _tc_dis(deg2).reshape(NPAD)

    z12 = _sc_spmm_panels(u12, src2, dst2, ew2, dis)
    s = _tc_sum(v3, z12, b1.reshape(1, F))
    t2 = _sc_spmm_single(s, src2, dst2, ew2, dis)
    h, y = _tc_head(s, t2, W20, W21, b2.reshape(1, D),
                    L1w, L1b.reshape(1, 128), L2w, L2b.reshape(1, P))
    return (y, h)
